# baseline (speedup 1.0000x reference)
import functools

import numpy as np
import jax
import jax.numpy as jnp
from jax.experimental import pallas as pl
from jax.experimental.pallas import tpu as pltpu

_ALPHA = 0.25
_LOSS_WEIGHT = 100.0
_IGNORE_INDEX = 255

_LOG2E = 1.4426950408889634
_LN2 = 0.6931471805599453


def _focal_body(C, Hb, D, W, H, inv_cmax, nsteps, pred_ref, tgt_ref, out_ref,
                vacc_ref, vcnt_ref):
    g = pl.program_id(0)
    step = g * pl.num_programs(1) + pl.program_id(1)

    @pl.when(step == 0)
    def _init():
        vacc_ref[...] = jnp.zeros((D, W), jnp.float32)
        vcnt_ref[...] = jnp.zeros((D, W), jnp.float32)

    xx = jax.lax.broadcasted_iota(jnp.int32, (D, W), 1).astype(jnp.float32)
    xsq = (xx - W / 2.0) * (xx - W / 2.0)
    h0 = (g * Hb).astype(jnp.float32) - H / 2.0

    vacc = jnp.zeros((D, W), jnp.float32)
    vcnt = jnp.zeros((D, W), jnp.float32)
    for hh in range(Hb):
        tgt_h = tgt_ref[0, hh]
        acc = jnp.zeros((D, W), jnp.float32)
        xt = jnp.zeros((D, W), jnp.float32)
        for c in range(C):
            x = pred_ref[0, c, hh]
            e = jnp.exp2(jnp.abs(x) * (-_LOG2E))
            t = 1.0 + e
            sp = jnp.maximum(x, 0.0) + jnp.log2(t) * _LN2
            inv = 1.0 / t
            sig = jnp.where(x >= 0.0, inv, e * inv)
            acc = acc + sp * (sig * sig)
            xt = jnp.where(tgt_h == c, x, xt)
        e = jnp.exp2(jnp.abs(xt) * (-_LOG2E))
        t = 1.0 + e
        sp = jnp.maximum(xt, 0.0) + jnp.log2(t) * _LN2
        inv = 1.0 / t
        sig = jnp.where(xt >= 0.0, inv, e * inv)
        oms = 1.0 - sig
        l1 = (sp - xt) * (_ALPHA * (oms * oms))
        l0t = sp * ((1.0 - _ALPHA) * (sig * sig))
        voxel = (1.0 - _ALPHA) * acc + (l1 - l0t)
        yy = h0 + hh
        w_row = jnp.sqrt(xsq + yy * yy) * inv_cmax + 1.0
        vis = tgt_h != _IGNORE_INDEX
        vacc = vacc + voxel * jnp.where(vis, w_row, 0.0)
        vcnt = vcnt + jnp.where(vis, 1.0, 0.0)

    vacc_ref[...] += vacc
    vcnt_ref[...] += vcnt

    @pl.when(step == nsteps - 1)
    def _flush():
        out_ref[0, 0] = jnp.sum(vacc_ref[...])
        out_ref[0, 1] = jnp.sum(vcnt_ref[...])


def kernel(pred, target):
    B, C, H, W, D = pred.shape
    Hb = 40
    nsteps = B * (H // Hb)

    predT = jnp.transpose(pred, (0, 1, 2, 4, 3))
    tgtT = jnp.transpose(target, (0, 1, 3, 2))

    inv_cmax = float(1.0 / np.sqrt(np.float32((H / 2.0) ** 2 + (W / 2.0) ** 2)))

    body = functools.partial(_focal_body, C, Hb, D, W, H, inv_cmax, nsteps)
    out = pl.pallas_call(
        body,
        grid=(H // Hb, B),
        in_specs=[
            pl.BlockSpec((1, C, Hb, D, W), lambda g, b: (b, 0, g, 0, 0)),
            pl.BlockSpec((1, Hb, D, W), lambda g, b: (b, g, 0, 0)),
        ],
        out_specs=pl.BlockSpec((1, 2), lambda g, b: (0, 0), memory_space=pltpu.SMEM),
        out_shape=jax.ShapeDtypeStruct((1, 2), jnp.float32),
        scratch_shapes=[
            pltpu.VMEM((D, W), jnp.float32),
            pltpu.VMEM((D, W), jnp.float32),
        ],
    )(predT, tgtT)
    return _LOSS_WEIGHT * out[0, 0] / out[0, 1]


# speedup vs baseline: 1.0142x; 1.0142x over previous
import functools

import numpy as np
import jax
import jax.numpy as jnp
from jax.experimental import pallas as pl
from jax.experimental.pallas import tpu as pltpu

_ALPHA = 0.25
_LOSS_WEIGHT = 100.0
_IGNORE_INDEX = 255

_LOG2E = 1.4426950408889634
_LN2 = 0.6931471805599453


def _focal_body(C, Hb, D, W, H, inv_cmax, nsteps, pred_ref, tgt_ref, out_ref,
                vacc_ref, vcnt_ref):
    g = pl.program_id(0)
    step = g * pl.num_programs(1) + pl.program_id(1)

    @pl.when(step == 0)
    def _init():
        vacc_ref[...] = jnp.zeros((D, W), jnp.float32)
        vcnt_ref[...] = jnp.zeros((D, W), jnp.float32)

    xx = jax.lax.broadcasted_iota(jnp.int32, (D, W), 1).astype(jnp.float32)
    xsq = (xx - W / 2.0) * (xx - W / 2.0)
    h0 = (g * Hb).astype(jnp.float32) - H / 2.0

    vacc = jnp.zeros((D, W), jnp.float32)
    vcnt = jnp.zeros((D, W), jnp.float32)
    for hh in range(Hb):
        tgt_h = tgt_ref[0, hh]
        acc = jnp.zeros((D, W), jnp.float32)
        xt = jnp.zeros((D, W), jnp.float32)
        for c in range(C):
            x = pred_ref[0, c, hh]
            e = jnp.exp2(jnp.abs(x) * (-_LOG2E))
            t = 1.0 + e
            sp = jnp.maximum(x, 0.0) + jnp.log2(t) * _LN2
            inv = 1.0 / t
            sig = jnp.where(x >= 0.0, inv, e * inv)
            acc = acc + sp * (sig * sig)
            xt = jnp.where(tgt_h == c, x, xt)
        e = jnp.exp2(jnp.abs(xt) * (-_LOG2E))
        t = 1.0 + e
        sp = jnp.maximum(xt, 0.0) + jnp.log2(t) * _LN2
        inv = 1.0 / t
        sig = jnp.where(xt >= 0.0, inv, e * inv)
        oms = 1.0 - sig
        l1 = (sp - xt) * (_ALPHA * (oms * oms))
        l0t = sp * ((1.0 - _ALPHA) * (sig * sig))
        voxel = (1.0 - _ALPHA) * acc + (l1 - l0t)
        yy = h0 + hh
        w_row = jnp.sqrt(xsq + yy * yy) * inv_cmax + 1.0
        vis = tgt_h != _IGNORE_INDEX
        vacc = vacc + voxel * jnp.where(vis, w_row, 0.0)
        vcnt = vcnt + jnp.where(vis, 1.0, 0.0)

    vacc_ref[...] += vacc
    vcnt_ref[...] += vcnt

    @pl.when(step == nsteps - 1)
    def _flush():
        out_ref[0, 0] = jnp.sum(vacc_ref[...])
        out_ref[0, 1] = jnp.sum(vcnt_ref[...])


def kernel(pred, target):
    B, C, H, W, D = pred.shape
    Hb = 20
    nsteps = B * (H // Hb)

    predT = jnp.transpose(pred, (0, 1, 2, 4, 3))
    tgtT = jnp.transpose(target, (0, 1, 3, 2))

    inv_cmax = float(1.0 / np.sqrt(np.float32((H / 2.0) ** 2 + (W / 2.0) ** 2)))

    body = functools.partial(_focal_body, C, Hb, D, W, H, inv_cmax, nsteps)
    out = pl.pallas_call(
        body,
        grid=(H // Hb, B),
        in_specs=[
            pl.BlockSpec((1, C, Hb, D, W), lambda g, b: (b, 0, g, 0, 0)),
            pl.BlockSpec((1, Hb, D, W), lambda g, b: (b, g, 0, 0)),
        ],
        out_specs=pl.BlockSpec((1, 2), lambda g, b: (0, 0), memory_space=pltpu.SMEM),
        out_shape=jax.ShapeDtypeStruct((1, 2), jnp.float32),
        scratch_shapes=[
            pltpu.VMEM((D, W), jnp.float32),
            pltpu.VMEM((D, W), jnp.float32),
        ],
    )(predT, tgtT)
    return _LOSS_WEIGHT * out[0, 0] / out[0, 1]
